# Initial kernel scaffold; baseline (speedup 1.0000x reference)
import jax
import jax.numpy as jnp
from jax.experimental import pallas as pl


def kernel(x, w_gate, b_gate, W1, b1, W2, b2, ln_gamma, ln_beta):
    raise NotImplementedError("write your pallas kernel here")


# speedup vs baseline: 1.8790x; 1.8790x over previous
import functools

import jax
import jax.numpy as jnp
from jax.experimental import pallas as pl

NE = 8
NB = 2
NS = 2048
ND = 1024
NH = 4096
NL = NS // 2
EPSG = 1e-06


def _routing_body(xb_ref, wg_ref, bg_ref, route_ref):
    logits = jnp.dot(xb_ref[...], wg_ref[...], preferred_element_type=jnp.float32)
    logits = logits + bg_ref[...]
    m = jnp.max(logits, axis=1, keepdims=True)
    ex = jnp.exp(logits - m)
    gate = ex / jnp.sum(ex, axis=1, keepdims=True)
    e_iota = jax.lax.broadcasted_iota(jnp.int32, (NB * NS, NE), 1)
    cnt = jnp.zeros((NB * NS, NE), dtype=jnp.float32)
    for f in range(NE):
        lf = logits[:, f : f + 1]
        beats = (lf > logits) | ((lf == logits) & (f < e_iota))
        cnt = cnt + beats.astype(jnp.float32)
    mask = (cnt < 2.0).astype(jnp.float32)
    masked = gate * mask
    m0 = masked[:NS, :]
    m1 = masked[NS:, :]
    denom = m0 + m1 + EPSG
    route_ref[:NS, :] = m0 / denom * 2.0
    route_ref[NS:, :] = m1 / denom * 2.0


def _select_body(rrow_ref, rcol_ref, sel_ref):
    rrow = rrow_ref[0]
    rcol = rcol_ref[0]
    s_iota = jax.lax.broadcasted_iota(jnp.int32, (256, NS), 1)
    rank = jnp.zeros((1, NS), dtype=jnp.float32)
    for tb in range(NS // 256):
        tcol = rcol[tb * 256 : (tb + 1) * 256, :]
        t_iota = jax.lax.broadcasted_iota(jnp.int32, (256, NS), 0) + tb * 256
        beats = (tcol > rrow) | ((tcol == rrow) & (t_iota < s_iota))
        rank = rank + jnp.sum(beats.astype(jnp.float32), axis=0, keepdims=True)
    sel_ref[0] = (rank < float(NL)).astype(jnp.float32)


def _mlp_body(xb_ref, w1_ref, w2_ref, b1_ref, b2_ref, sel_ref, out_ref):
    e = pl.program_id(0)
    hc = pl.program_id(1)
    rb = pl.program_id(2)

    @pl.when((e == 0) & (hc == 0) & (rb == 0))
    def _():
        out_ref[...] = jnp.zeros_like(out_ref)

    xb = xb_ref[...]
    h = jnp.dot(xb, w1_ref[0], preferred_element_type=jnp.float32)
    h = h + b1_ref[0, 0]
    h = 0.5 * h * (1.0 + jax.lax.erf(h * 0.7071067811865476))
    part = jnp.dot(h.astype(jnp.bfloat16), w2_ref[0], preferred_element_type=jnp.float32)
    contrib = jnp.where(hc == 0, part + b2_ref[0, 0], part)
    selc = sel_ref[0]
    out_ref[pl.ds(rb * 256, 256), :] += selc * contrib


def _ln_body(o_ref, x_ref, g_ref, b_ref, out_ref):
    z = o_ref[...] + x_ref[...]
    mean = jnp.mean(z, axis=-1, keepdims=True)
    zc = z - mean
    var = jnp.mean(zc * zc, axis=-1, keepdims=True)
    out_ref[...] = zc * jax.lax.rsqrt(var + 1e-05) * g_ref[...] + b_ref[...]


@functools.partial(jax.jit, static_argnums=())
def kernel(x, w_gate, b_gate, W1, b1, W2, b2, ln_gamma, ln_beta):
    B, S, D = x.shape
    H = W1.shape[2]
    xf = x.reshape(B * S, D)
    xb16 = xf.astype(jnp.bfloat16)

    route = pl.pallas_call(
        _routing_body,
        out_shape=jax.ShapeDtypeStruct((B * S, NE), jnp.float32),
    )(xb16, w_gate.astype(jnp.bfloat16), b_gate.reshape(1, NE))

    route_eb = jnp.transpose(route).reshape(NE * B, S)
    rrow = route_eb.reshape(NE * B, 1, S)
    rcol = route_eb.reshape(NE * B, S, 1)

    sel = pl.pallas_call(
        _select_body,
        out_shape=jax.ShapeDtypeStruct((NE * B, 1, S), jnp.float32),
        grid=(NE * B,),
        in_specs=[
            pl.BlockSpec((1, 1, S), lambda i: (i, 0, 0)),
            pl.BlockSpec((1, S, 1), lambda i: (i, 0, 0)),
        ],
        out_specs=pl.BlockSpec((1, 1, S), lambda i: (i, 0, 0)),
    )(rrow, rcol)

    sel_col = sel.reshape(NE, B * S, 1)

    NRB = (B * S) // 256
    outs = pl.pallas_call(
        _mlp_body,
        out_shape=jax.ShapeDtypeStruct((B * S, D), jnp.float32),
        grid=(NE, 2, NRB),
        in_specs=[
            pl.BlockSpec((256, D), lambda e, hc, rb: (rb, 0)),
            pl.BlockSpec((1, D, H // 2), lambda e, hc, rb: (e, 0, hc)),
            pl.BlockSpec((1, H // 2, D), lambda e, hc, rb: (e, hc, 0)),
            pl.BlockSpec((1, 1, H // 2), lambda e, hc, rb: (e, 0, hc)),
            pl.BlockSpec((1, 1, D), lambda e, hc, rb: (e, 0, 0)),
            pl.BlockSpec((1, 256, 1), lambda e, hc, rb: (e, rb, 0)),
        ],
        out_specs=pl.BlockSpec((B * S, D), lambda e, hc, rb: (0, 0)),
    )(
        xb16,
        W1.astype(jnp.bfloat16),
        W2.astype(jnp.bfloat16),
        b1.reshape(NE, 1, H),
        b2.reshape(NE, 1, D),
        sel_col,
    )

    out_ln = pl.pallas_call(
        _ln_body,
        out_shape=jax.ShapeDtypeStruct((B * S, D), jnp.float32),
        grid=(8,),
        in_specs=[
            pl.BlockSpec(((B * S) // 8, D), lambda i: (i, 0)),
            pl.BlockSpec(((B * S) // 8, D), lambda i: (i, 0)),
            pl.BlockSpec((1, D), lambda i: (0, 0)),
            pl.BlockSpec((1, D), lambda i: (0, 0)),
        ],
        out_specs=pl.BlockSpec(((B * S) // 8, D), lambda i: (i, 0)),
    )(outs, xf, ln_gamma.reshape(1, D), ln_beta.reshape(1, D))
    return out_ln.reshape(B, S, D)
